# baseline (speedup 1.0000x reference)
import dataclasses
import functools

import jax
import jax.numpy as jnp
from jax import lax
from jax.experimental import pallas as pl
from jax.experimental.pallas import tpu as pltpu
from jax.experimental.pallas import tpu_sc as plsc

_HIDDEN = 768
_HALF = _HIDDEN // 2
_EPS = 1e-12
_NC = 2
_NS = 16
_NW = _NC * _NS
_LANES = 16
_CHUNK = 64
_TBLK = 512
_SC_SPLITS = (1024, 1024)
_TC_SPLITS = (512, 512, 1024)


def _sc_gather_pack_chunk(word_table, idx_flat, chunk_base, n_rows, s, s_ch):
    per_w = n_rows // _NW
    n_sub = per_w // _CHUNK
    w_per_b = s_ch // per_w
    mesh = plsc.VectorSubcoreMesh(core_axis_name="c", subcore_axis_name="s")
    cp = pltpu.CompilerParams()
    if "needs_layout_passes" in pltpu.CompilerParams.__dataclass_fields__:
        cp = dataclasses.replace(cp, needs_layout_passes=False)

    @functools.partial(
        pl.kernel,
        compiler_params=cp,
        out_type=jax.ShapeDtypeStruct((n_rows, _HALF), jnp.int32),
        mesh=mesh,
        scratch_types=[
            pltpu.VMEM((per_w,), jnp.int32),
            pltpu.VMEM((_CHUNK, _HIDDEN), jnp.float32),
            pltpu.VMEM((_CHUNK, _HIDDEN), jnp.float32),
            pltpu.VMEM((_CHUNK, _HALF), jnp.int32),
            pltpu.SemaphoreType.DMA,
            pltpu.SemaphoreType.DMA,
        ],
    )
    def gather_kernel(table_hbm, idx_hbm, out_hbm, idx_v, rows_a, rows_b,
                      packed_v, sem_a, sem_b):
        wid = lax.axis_index("s") * _NC + lax.axis_index("c")
        b = wid // w_per_b
        j = (wid % w_per_b) * per_w
        src = b * s + chunk_base + j
        pltpu.sync_copy(idx_hbm.at[pl.ds(src, per_w)], idx_v)

        base = wid * per_w
        bufs = (rows_a, rows_b)
        sems = (sem_a, sem_b)

        def pack_and_flush(rows_v, c):
            @pl.loop(0, _CHUNK)
            def _(r):
                for g in range(_HALF // _LANES):
                    lo = plsc.bitcast(
                        rows_v[r, pl.ds(g * _LANES, _LANES)], jnp.uint32)
                    hi = plsc.bitcast(
                        rows_v[r, pl.ds(_HALF + g * _LANES, _LANES)], jnp.uint32)
                    word = (lo >> jnp.uint32(16)) | (hi & jnp.uint32(0xFFFF0000))
                    packed_v[r, pl.ds(g * _LANES, _LANES)] = plsc.bitcast(
                        word, jnp.int32)
            pltpu.sync_copy(packed_v, out_hbm.at[pl.ds(base + c * _CHUNK, _CHUNK)])

        copies = [None] * n_sub
        copies[0] = pltpu.async_copy(
            table_hbm.at[idx_v.at[pl.ds(0, _CHUNK)]], bufs[0], sems[0]
        )
        for c in range(n_sub):
            if c + 1 < n_sub:
                copies[c + 1] = pltpu.async_copy(
                    table_hbm.at[idx_v.at[pl.ds((c + 1) * _CHUNK, _CHUNK)]],
                    bufs[(c + 1) % 2],
                    sems[(c + 1) % 2],
                )
            copies[c].wait()
            pack_and_flush(bufs[c % 2], c)

    return gather_kernel(word_table, idx_flat)


def _ln_body(g_ref, p_ref, tt_ref, ty_ref, w_ref, b_ref, o_ref):
    gi = g_ref[0]
    xlo = lax.bitcast_convert_type(gi << 16, jnp.float32)
    xhi = lax.bitcast_convert_type(gi & jnp.int32(-65536), jnp.float32)
    x = jnp.concatenate([xlo, xhi], axis=1) + p_ref[...]
    t0 = ty_ref[0:1, :]
    t1 = ty_ref[1:2, :]
    tt = tt_ref[0].astype(jnp.float32)
    x = x + t0 + tt * (t1 - t0)
    mean = jnp.mean(x, axis=1, keepdims=True)
    xc = x - mean
    var = jnp.mean(xc * xc, axis=1, keepdims=True)
    y = xc * lax.rsqrt(var + _EPS)
    o_ref[0] = y * w_ref[...] + b_ref[...]


def _ln_body_acc(g_ref, p_ref, tt_ref, ty_ref, w_ref, b_ref, _buf_ref, o_ref):
    _ln_body(g_ref, p_ref, tt_ref, ty_ref, w_ref, b_ref, o_ref)


def _tc_add_ln_chunk(g_h, pos_table, tt3, type_table, ln_w, ln_b, buf,
                     in_start, out_start, size, b, s):
    n_blk = size // _TBLK
    qin = in_start // _TBLK
    qout = out_start // _TBLK
    grid = (n_blk, b)
    in_specs = [
        pl.BlockSpec((1, _TBLK, _HALF), lambda i, j, q=qin: (j, q + i, 0)),
        pl.BlockSpec((_TBLK, _HIDDEN), lambda i, j, q=qout: (q + i, 0)),
        pl.BlockSpec((1, _TBLK, 1), lambda i, j, q=qout: (j, q + i, 0)),
        pl.BlockSpec((2, _HIDDEN), lambda i, j: (0, 0)),
        pl.BlockSpec((1, _HIDDEN), lambda i, j: (0, 0)),
        pl.BlockSpec((1, _HIDDEN), lambda i, j: (0, 0)),
    ]
    args = [g_h, pos_table, tt3, type_table, ln_w, ln_b]
    body = _ln_body
    aliases = {}
    if buf is not None:
        in_specs.append(pl.BlockSpec(memory_space=pltpu.MemorySpace.HBM))
        args.append(buf)
        aliases = {6: 0}
        body = _ln_body_acc
    return pl.pallas_call(
        body,
        grid=grid,
        in_specs=in_specs,
        out_specs=pl.BlockSpec((1, _TBLK, _HIDDEN), lambda i, j, q=qout: (j, q + i, 0)),
        out_shape=jax.ShapeDtypeStruct((b, s, _HIDDEN), jnp.float32),
        input_output_aliases=aliases,
    )(*args)


def kernel(input_ids, token_type_ids, word_table, pos_table, type_table, ln_weight, ln_bias):
    b, s = input_ids.shape
    ln_w = ln_weight.reshape(1, -1)
    ln_b = ln_bias.reshape(1, -1)
    idx_flat = input_ids.reshape(-1).astype(jnp.int32)
    tt3 = token_type_ids.reshape(b, s, 1)

    gathered = []
    base = 0
    for s_ch in _SC_SPLITS:
        g_h = _sc_gather_pack_chunk(word_table, idx_flat, base, b * s_ch, s, s_ch)
        gathered.append((base, s_ch, g_h.reshape(b, s_ch, _HALF)))
        base += s_ch

    buf = None
    out_start = 0
    for size in _TC_SPLITS:
        sc_start, sc_size, g_h = next(
            (st, sz, g) for (st, sz, g) in gathered
            if st <= out_start and out_start + size <= st + sz
        )
        buf = _tc_add_ln_chunk(
            g_h, pos_table, tt3, type_table, ln_w, ln_b, buf,
            out_start - sc_start, out_start, size, b, s,
        )
        out_start += size
    return buf


# speedup vs baseline: 1.2313x; 1.2313x over previous
import dataclasses
import functools

import jax
import jax.numpy as jnp
from jax import lax
from jax.experimental import pallas as pl
from jax.experimental.pallas import tpu as pltpu
from jax.experimental.pallas import tpu_sc as plsc

_HIDDEN = 768
_HALF = _HIDDEN // 2
_EPS = 1e-12
_NC = 2
_NS = 16
_NW = _NC * _NS
_LANES = 16
_CHUNK = 64
_TBLK = 512
_SC_SPLITS = (1024, 1024)
_TC_SPLITS = (512, 512, 1024)


def _sc_gather_pack_chunk(word_table, idx_flat, chunk_base, n_rows, s, s_ch):
    per_w = n_rows // _NW
    n_sub = per_w // _CHUNK
    w_per_b = s_ch // per_w
    mesh = plsc.VectorSubcoreMesh(core_axis_name="c", subcore_axis_name="s")
    cp = pltpu.CompilerParams()
    if "needs_layout_passes" in pltpu.CompilerParams.__dataclass_fields__:
        cp = dataclasses.replace(cp, needs_layout_passes=False)

    @functools.partial(
        pl.kernel,
        compiler_params=cp,
        out_type=jax.ShapeDtypeStruct((n_rows, _HALF), jnp.int32),
        mesh=mesh,
        scratch_types=[
            pltpu.VMEM((per_w,), jnp.int32),
            pltpu.VMEM((_CHUNK, _HIDDEN), jnp.float32),
            pltpu.VMEM((_CHUNK, _HIDDEN), jnp.float32),
            pltpu.VMEM((_CHUNK, _HALF), jnp.int32),
            pltpu.SemaphoreType.DMA,
            pltpu.SemaphoreType.DMA,
        ],
    )
    def gather_kernel(table_hbm, idx_hbm, out_hbm, idx_v, rows_a, rows_b,
                      packed_v, sem_a, sem_b):
        wid = lax.axis_index("s") * _NC + lax.axis_index("c")
        b = wid // w_per_b
        j = (wid % w_per_b) * per_w
        src = b * s + chunk_base + j
        pltpu.sync_copy(idx_hbm.at[pl.ds(src, per_w)], idx_v)

        base = wid * per_w
        bufs = (rows_a, rows_b)
        sems = (sem_a, sem_b)

        def pack_and_flush(rows_v, c):
            @plsc.parallel_loop(0, _CHUNK, 1, unroll=8)
            def _(r):
                for g in range(_HALF // _LANES):
                    lo = plsc.bitcast(
                        rows_v[r, pl.ds(g * _LANES, _LANES)], jnp.uint32)
                    hi = plsc.bitcast(
                        rows_v[r, pl.ds(_HALF + g * _LANES, _LANES)], jnp.uint32)
                    word = (lo >> jnp.uint32(16)) | (hi & jnp.uint32(0xFFFF0000))
                    packed_v[r, pl.ds(g * _LANES, _LANES)] = plsc.bitcast(
                        word, jnp.int32)
            pltpu.sync_copy(packed_v, out_hbm.at[pl.ds(base + c * _CHUNK, _CHUNK)])

        copies = [None] * n_sub
        copies[0] = pltpu.async_copy(
            table_hbm.at[idx_v.at[pl.ds(0, _CHUNK)]], bufs[0], sems[0]
        )
        for c in range(n_sub):
            if c + 1 < n_sub:
                copies[c + 1] = pltpu.async_copy(
                    table_hbm.at[idx_v.at[pl.ds((c + 1) * _CHUNK, _CHUNK)]],
                    bufs[(c + 1) % 2],
                    sems[(c + 1) % 2],
                )
            copies[c].wait()
            pack_and_flush(bufs[c % 2], c)

    return gather_kernel(word_table, idx_flat)


def _ln_body(g_ref, p_ref, tt_ref, ty_ref, w_ref, b_ref, o_ref):
    gi = g_ref[0]
    xlo = lax.bitcast_convert_type(gi << 16, jnp.float32)
    xhi = lax.bitcast_convert_type(gi & jnp.int32(-65536), jnp.float32)
    x = jnp.concatenate([xlo, xhi], axis=1) + p_ref[...]
    t0 = ty_ref[0:1, :]
    t1 = ty_ref[1:2, :]
    tt = tt_ref[0].astype(jnp.float32)
    x = x + t0 + tt * (t1 - t0)
    mean = jnp.mean(x, axis=1, keepdims=True)
    xc = x - mean
    var = jnp.mean(xc * xc, axis=1, keepdims=True)
    y = xc * lax.rsqrt(var + _EPS)
    o_ref[0] = y * w_ref[...] + b_ref[...]


def _ln_body_acc(g_ref, p_ref, tt_ref, ty_ref, w_ref, b_ref, _buf_ref, o_ref):
    _ln_body(g_ref, p_ref, tt_ref, ty_ref, w_ref, b_ref, o_ref)


def _tc_add_ln_chunk(g_h, pos_table, tt3, type_table, ln_w, ln_b, buf,
                     in_start, out_start, size, b, s):
    n_blk = size // _TBLK
    qin = in_start // _TBLK
    qout = out_start // _TBLK
    grid = (n_blk, b)
    in_specs = [
        pl.BlockSpec((1, _TBLK, _HALF), lambda i, j, q=qin: (j, q + i, 0)),
        pl.BlockSpec((_TBLK, _HIDDEN), lambda i, j, q=qout: (q + i, 0)),
        pl.BlockSpec((1, _TBLK, 1), lambda i, j, q=qout: (j, q + i, 0)),
        pl.BlockSpec((2, _HIDDEN), lambda i, j: (0, 0)),
        pl.BlockSpec((1, _HIDDEN), lambda i, j: (0, 0)),
        pl.BlockSpec((1, _HIDDEN), lambda i, j: (0, 0)),
    ]
    args = [g_h, pos_table, tt3, type_table, ln_w, ln_b]
    body = _ln_body
    aliases = {}
    if buf is not None:
        in_specs.append(pl.BlockSpec(memory_space=pltpu.MemorySpace.HBM))
        args.append(buf)
        aliases = {6: 0}
        body = _ln_body_acc
    return pl.pallas_call(
        body,
        grid=grid,
        in_specs=in_specs,
        out_specs=pl.BlockSpec((1, _TBLK, _HIDDEN), lambda i, j, q=qout: (j, q + i, 0)),
        out_shape=jax.ShapeDtypeStruct((b, s, _HIDDEN), jnp.float32),
        input_output_aliases=aliases,
    )(*args)


def kernel(input_ids, token_type_ids, word_table, pos_table, type_table, ln_weight, ln_bias):
    b, s = input_ids.shape
    ln_w = ln_weight.reshape(1, -1)
    ln_b = ln_bias.reshape(1, -1)
    idx_flat = input_ids.reshape(-1).astype(jnp.int32)
    tt3 = token_type_ids.reshape(b, s, 1)

    gathered = []
    base = 0
    for s_ch in _SC_SPLITS:
        g_h = _sc_gather_pack_chunk(word_table, idx_flat, base, b * s_ch, s, s_ch)
        gathered.append((base, s_ch, g_h.reshape(b, s_ch, _HALF)))
        base += s_ch

    buf = None
    out_start = 0
    for size in _TC_SPLITS:
        sc_start, sc_size, g_h = next(
            (st, sz, g) for (st, sz, g) in gathered
            if st <= out_start and out_start + size <= st + sz
        )
        buf = _tc_add_ln_chunk(
            g_h, pos_table, tt3, type_table, ln_w, ln_b, buf,
            out_start - sc_start, out_start, size, b, s,
        )
        out_start += size
    return buf


# speedup vs baseline: 1.3630x; 1.1070x over previous
import functools

import jax
import jax.numpy as jnp
from jax import lax
from jax.experimental import pallas as pl
from jax.experimental.pallas import tpu as pltpu
from jax.experimental.pallas import tpu_sc as plsc

_HIDDEN = 768
_EPS = 1e-12
_NC = 2
_NS = 16
_NW = _NC * _NS
_CHUNK = 128
_TBLK = 512
_SC_SPLITS = (1024, 1024)
_TC_SPLITS = (512, 512, 1024)


def _sc_gather_chunk(word_table, idx_flat, chunk_base, n_rows, s, s_ch):
    per_w = n_rows // _NW
    n_sub = per_w // _CHUNK
    n_buf = min(2, n_sub)
    w_per_b = s_ch // per_w
    mesh = plsc.VectorSubcoreMesh(core_axis_name="c", subcore_axis_name="s")

    @functools.partial(
        pl.kernel,
        out_type=jax.ShapeDtypeStruct((n_rows, _HIDDEN), jnp.float32),
        mesh=mesh,
        scratch_types=(
            [pltpu.VMEM((per_w,), jnp.int32)]
            + [pltpu.VMEM((_CHUNK, _HIDDEN), jnp.float32)] * n_buf
            + [pltpu.SemaphoreType.DMA] * n_buf
        ),
    )
    def gather_kernel(table_hbm, idx_hbm, out_hbm, idx_v, *bufs_sems):
        bufs, sems = bufs_sems[:n_buf], bufs_sems[n_buf:]
        wid = lax.axis_index("s") * _NC + lax.axis_index("c")
        b = wid // w_per_b
        j = (wid % w_per_b) * per_w
        src = b * s + chunk_base + j
        pltpu.sync_copy(idx_hbm.at[pl.ds(src, per_w)], idx_v)

        base = wid * per_w
        copies = [None] * n_sub
        copies[0] = pltpu.async_copy(
            table_hbm.at[idx_v.at[pl.ds(0, _CHUNK)]], bufs[0], sems[0]
        )
        for c in range(n_sub):
            if c + 1 < n_sub:
                copies[c + 1] = pltpu.async_copy(
                    table_hbm.at[idx_v.at[pl.ds((c + 1) * _CHUNK, _CHUNK)]],
                    bufs[(c + 1) % n_buf],
                    sems[(c + 1) % n_buf],
                )
            copies[c].wait()
            pltpu.sync_copy(bufs[c % n_buf], out_hbm.at[pl.ds(base + c * _CHUNK, _CHUNK)])

    return gather_kernel(word_table, idx_flat)


def _ln_body(g_ref, p_ref, tt_ref, ty_ref, w_ref, b_ref, o_ref):
    x = g_ref[0] + p_ref[...]
    t0 = ty_ref[0:1, :]
    t1 = ty_ref[1:2, :]
    tt = tt_ref[0].astype(jnp.float32)
    x = x + t0 + tt * (t1 - t0)
    mean = jnp.mean(x, axis=1, keepdims=True)
    xc = x - mean
    var = jnp.mean(xc * xc, axis=1, keepdims=True)
    y = xc * lax.rsqrt(var + _EPS)
    o_ref[0] = y * w_ref[...] + b_ref[...]


def _ln_body_acc(g_ref, p_ref, tt_ref, ty_ref, w_ref, b_ref, _buf_ref, o_ref):
    _ln_body(g_ref, p_ref, tt_ref, ty_ref, w_ref, b_ref, o_ref)


def _tc_add_ln_chunk(g_h, pos_table, tt3, type_table, ln_w, ln_b, buf,
                     in_start, out_start, size, b, s):
    n_blk = size // _TBLK
    qin = in_start // _TBLK
    qout = out_start // _TBLK
    grid = (n_blk, b)
    in_specs = [
        pl.BlockSpec((1, _TBLK, _HIDDEN), lambda i, j, q=qin: (j, q + i, 0)),
        pl.BlockSpec((_TBLK, _HIDDEN), lambda i, j, q=qout: (q + i, 0)),
        pl.BlockSpec((1, _TBLK, 1), lambda i, j, q=qout: (j, q + i, 0)),
        pl.BlockSpec((2, _HIDDEN), lambda i, j: (0, 0)),
        pl.BlockSpec((1, _HIDDEN), lambda i, j: (0, 0)),
        pl.BlockSpec((1, _HIDDEN), lambda i, j: (0, 0)),
    ]
    args = [g_h, pos_table, tt3, type_table, ln_w, ln_b]
    body = _ln_body
    aliases = {}
    if buf is not None:
        in_specs.append(pl.BlockSpec(memory_space=pltpu.MemorySpace.HBM))
        args.append(buf)
        aliases = {6: 0}
        body = _ln_body_acc
    return pl.pallas_call(
        body,
        grid=grid,
        in_specs=in_specs,
        out_specs=pl.BlockSpec((1, _TBLK, _HIDDEN), lambda i, j, q=qout: (j, q + i, 0)),
        out_shape=jax.ShapeDtypeStruct((b, s, _HIDDEN), jnp.float32),
        input_output_aliases=aliases,
    )(*args)


def kernel(input_ids, token_type_ids, word_table, pos_table, type_table, ln_weight, ln_bias):
    b, s = input_ids.shape
    ln_w = ln_weight.reshape(1, -1)
    ln_b = ln_bias.reshape(1, -1)
    idx_flat = input_ids.reshape(-1).astype(jnp.int32)
    tt3 = token_type_ids.reshape(b, s, 1)

    gathered = []
    base = 0
    for s_ch in _SC_SPLITS:
        g_h = _sc_gather_chunk(word_table, idx_flat, base, b * s_ch, s, s_ch)
        gathered.append((base, s_ch, g_h.reshape(b, s_ch, _HIDDEN)))
        base += s_ch

    buf = None
    out_start = 0
    for size in _TC_SPLITS:
        sc_start, sc_size, g_h = next(
            (st, sz, g) for (st, sz, g) in gathered
            if st <= out_start and out_start + size <= st + sz
        )
        buf = _tc_add_ln_chunk(
            g_h, pos_table, tt3, type_table, ln_w, ln_b, buf,
            out_start - sc_start, out_start, size, b, s,
        )
        out_start += size
    return buf


# speedup vs baseline: 1.3775x; 1.0106x over previous
import functools

import jax
import jax.numpy as jnp
from jax import lax
from jax.experimental import pallas as pl
from jax.experimental.pallas import tpu as pltpu
from jax.experimental.pallas import tpu_sc as plsc

_HIDDEN = 768
_EPS = 1e-12
_NC = 2
_NS = 16
_NW = _NC * _NS
_CHUNK = 128
_TBLK = 512
_SC_SPLITS = (1024, 1024)
_TC_SPLITS = (512, 512, 1024)


def _sc_gather_chunk(word_table, idx_flat, chunk_base, n_rows, s, s_ch):
    per_w = n_rows // _NW
    n_sub = per_w // _CHUNK
    n_buf = min(2, n_sub)
    w_per_b = s_ch // per_w
    mesh = plsc.VectorSubcoreMesh(core_axis_name="c", subcore_axis_name="s")

    @functools.partial(
        pl.kernel,
        out_type=jax.ShapeDtypeStruct((n_rows, _HIDDEN), jnp.float32),
        mesh=mesh,
        scratch_types=(
            [pltpu.VMEM((per_w,), jnp.int32)]
            + [pltpu.VMEM((_CHUNK, _HIDDEN), jnp.float32)] * n_buf
            + [pltpu.SemaphoreType.DMA] * n_buf
        ),
    )
    def gather_kernel(table_hbm, idx_hbm, out_hbm, idx_v, *bufs_sems):
        bufs, sems = bufs_sems[:n_buf], bufs_sems[n_buf:]
        wid = lax.axis_index("s") * _NC + lax.axis_index("c")
        b = wid // w_per_b
        j = (wid % w_per_b) * per_w
        pltpu.sync_copy(idx_hbm.at[b, pl.ds(chunk_base + j, per_w)], idx_v)

        base = wid * per_w
        copies = [None] * n_sub
        copies[0] = pltpu.async_copy(
            table_hbm.at[idx_v.at[pl.ds(0, _CHUNK)]], bufs[0], sems[0]
        )
        for c in range(n_sub):
            if c + 1 < n_sub:
                copies[c + 1] = pltpu.async_copy(
                    table_hbm.at[idx_v.at[pl.ds((c + 1) * _CHUNK, _CHUNK)]],
                    bufs[(c + 1) % n_buf],
                    sems[(c + 1) % n_buf],
                )
            copies[c].wait()
            pltpu.sync_copy(bufs[c % n_buf], out_hbm.at[pl.ds(base + c * _CHUNK, _CHUNK)])

    return gather_kernel(word_table, idx_flat)


def _ln_body(g_ref, p_ref, tt_ref, ty_ref, w_ref, b_ref, o_ref):
    x = g_ref[0] + p_ref[...]
    t0 = ty_ref[0:1, :]
    t1 = ty_ref[1:2, :]
    tt = tt_ref[0].astype(jnp.float32)
    x = x + t0 + tt * (t1 - t0)
    mean = jnp.mean(x, axis=1, keepdims=True)
    xc = x - mean
    var = jnp.mean(xc * xc, axis=1, keepdims=True)
    y = xc * lax.rsqrt(var + _EPS)
    o_ref[0] = y * w_ref[...] + b_ref[...]


def _ln_body_acc(g_ref, p_ref, tt_ref, ty_ref, w_ref, b_ref, _buf_ref, o_ref):
    _ln_body(g_ref, p_ref, tt_ref, ty_ref, w_ref, b_ref, o_ref)


def _tc_add_ln_chunk(g_h, pos_table, tt3, type_table, ln_w, ln_b, buf,
                     in_start, out_start, size, b, s):
    n_blk = size // _TBLK
    qin = in_start // _TBLK
    qout = out_start // _TBLK
    grid = (n_blk, b)
    in_specs = [
        pl.BlockSpec((1, _TBLK, _HIDDEN), lambda i, j, q=qin: (j, q + i, 0)),
        pl.BlockSpec((_TBLK, _HIDDEN), lambda i, j, q=qout: (q + i, 0)),
        pl.BlockSpec((1, _TBLK, 1), lambda i, j, q=qout: (j, q + i, 0)),
        pl.BlockSpec((2, _HIDDEN), lambda i, j: (0, 0)),
        pl.BlockSpec((1, _HIDDEN), lambda i, j: (0, 0)),
        pl.BlockSpec((1, _HIDDEN), lambda i, j: (0, 0)),
    ]
    args = [g_h, pos_table, tt3, type_table, ln_w, ln_b]
    body = _ln_body
    aliases = {}
    if buf is not None:
        in_specs.append(pl.BlockSpec(memory_space=pltpu.MemorySpace.HBM))
        args.append(buf)
        aliases = {6: 0}
        body = _ln_body_acc
    return pl.pallas_call(
        body,
        grid=grid,
        in_specs=in_specs,
        out_specs=pl.BlockSpec((1, _TBLK, _HIDDEN), lambda i, j, q=qout: (j, q + i, 0)),
        out_shape=jax.ShapeDtypeStruct((b, s, _HIDDEN), jnp.float32),
        input_output_aliases=aliases,
    )(*args)


def kernel(input_ids, token_type_ids, word_table, pos_table, type_table, ln_weight, ln_bias):
    b, s = input_ids.shape
    ln_w = ln_weight.reshape(1, -1)
    ln_b = ln_bias.reshape(1, -1)
    idx2d = input_ids.astype(jnp.int32)
    tt3 = token_type_ids.reshape(b, s, 1)

    gathered = []
    base = 0
    for s_ch in _SC_SPLITS:
        g_h = _sc_gather_chunk(word_table, idx2d, base, b * s_ch, s, s_ch)
        gathered.append((base, s_ch, g_h.reshape(b, s_ch, _HIDDEN)))
        base += s_ch

    buf = None
    out_start = 0
    for size in _TC_SPLITS:
        sc_start, sc_size, g_h = next(
            (st, sz, g) for (st, sz, g) in gathered
            if st <= out_start and out_start + size <= st + sz
        )
        buf = _tc_add_ln_chunk(
            g_h, pos_table, tt3, type_table, ln_w, ln_b, buf,
            out_start - sc_start, out_start, size, b, s,
        )
        out_start += size
    return buf


# speedup vs baseline: 1.3955x; 1.0130x over previous
import functools

import jax
import jax.numpy as jnp
from jax import lax
from jax.experimental import pallas as pl
from jax.experimental.pallas import tpu as pltpu
from jax.experimental.pallas import tpu_sc as plsc

_HIDDEN = 768
_EPS = 1e-12
_NC = 2
_NS = 16
_NW = _NC * _NS
_CHUNK = 128
_TBLK = 512
_SC_SPLITS = (1024, 1024)
_TC_SPLITS = (512, 512, 1024)


def _sc_gather_chunk(word_table, idx_flat, chunk_base, n_rows, s, s_ch):
    per_w = n_rows // _NW
    n_sub = per_w // _CHUNK
    n_buf = min(2, n_sub)
    w_per_b = s_ch // per_w
    mesh = plsc.VectorSubcoreMesh(core_axis_name="c", subcore_axis_name="s")

    @functools.partial(
        pl.kernel,
        out_type=jax.ShapeDtypeStruct((n_rows, _HIDDEN), jnp.float32),
        mesh=mesh,
        scratch_types=(
            [pltpu.VMEM((per_w,), jnp.int32)]
            + [pltpu.VMEM((_CHUNK, _HIDDEN), jnp.float32)] * n_buf
            + [pltpu.SemaphoreType.DMA] * n_buf
        ),
    )
    def gather_kernel(table_hbm, idx_hbm, out_hbm, idx_v, *bufs_sems):
        bufs, sems = bufs_sems[:n_buf], bufs_sems[n_buf:]
        wid = lax.axis_index("s") * _NC + lax.axis_index("c")
        b = wid // w_per_b
        j = (wid % w_per_b) * per_w
        pltpu.sync_copy(idx_hbm.at[b, pl.ds(chunk_base + j, per_w)], idx_v)

        base = wid * per_w
        copies = [None] * n_sub
        copies[0] = pltpu.async_copy(
            table_hbm.at[idx_v.at[pl.ds(0, _CHUNK)]], bufs[0], sems[0]
        )
        for c in range(n_sub):
            if c + 1 < n_sub:
                copies[c + 1] = pltpu.async_copy(
                    table_hbm.at[idx_v.at[pl.ds((c + 1) * _CHUNK, _CHUNK)]],
                    bufs[(c + 1) % n_buf],
                    sems[(c + 1) % n_buf],
                )
            copies[c].wait()
            pltpu.sync_copy(bufs[c % n_buf], out_hbm.at[pl.ds(base + c * _CHUNK, _CHUNK)])

    return gather_kernel(word_table, idx_flat)


def _ln_body(g_ref, p_ref, tt_ref, ty_ref, w_ref, b_ref, o_ref):
    x = g_ref[0] + p_ref[...]
    t0 = ty_ref[0:1, :]
    t1 = ty_ref[1:2, :]
    tt = tt_ref[0].astype(jnp.float32)
    x = x + t0 + tt * (t1 - t0)
    mean = jnp.mean(x, axis=1, keepdims=True)
    xc = x - mean
    var = jnp.mean(xc * xc, axis=1, keepdims=True)
    y = xc * lax.rsqrt(var + _EPS)
    o_ref[0] = y * w_ref[...] + b_ref[...]


def _ln_body_acc(g_ref, p_ref, tt_ref, ty_ref, w_ref, b_ref, _buf_ref, o_ref):
    _ln_body(g_ref, p_ref, tt_ref, ty_ref, w_ref, b_ref, o_ref)


def _tc_add_ln_chunk(g_h, pos_table, tt3, type_table, ln_w, ln_b, buf,
                     in_start, out_start, size, b, s):
    tblk = min(size, 1024)
    n_blk = size // tblk
    qin = in_start // tblk
    qout = out_start // tblk
    grid = (n_blk, b)
    in_specs = [
        pl.BlockSpec((1, tblk, _HIDDEN), lambda i, j, q=qin: (j, q + i, 0)),
        pl.BlockSpec((tblk, _HIDDEN), lambda i, j, q=qout: (q + i, 0)),
        pl.BlockSpec((1, tblk, 1), lambda i, j, q=qout: (j, q + i, 0)),
        pl.BlockSpec((2, _HIDDEN), lambda i, j: (0, 0)),
        pl.BlockSpec((1, _HIDDEN), lambda i, j: (0, 0)),
        pl.BlockSpec((1, _HIDDEN), lambda i, j: (0, 0)),
    ]
    args = [g_h, pos_table, tt3, type_table, ln_w, ln_b]
    body = _ln_body
    aliases = {}
    if buf is not None:
        in_specs.append(pl.BlockSpec(memory_space=pltpu.MemorySpace.HBM))
        args.append(buf)
        aliases = {6: 0}
        body = _ln_body_acc
    return pl.pallas_call(
        body,
        grid=grid,
        in_specs=in_specs,
        out_specs=pl.BlockSpec(
            (1, tblk, _HIDDEN), lambda i, j, q=qout: (j, q + i, 0)),
        out_shape=jax.ShapeDtypeStruct((b, s, _HIDDEN), jnp.float32),
        input_output_aliases=aliases,
    )(*args)


def kernel(input_ids, token_type_ids, word_table, pos_table, type_table, ln_weight, ln_bias):
    b, s = input_ids.shape
    ln_w = ln_weight.reshape(1, -1)
    ln_b = ln_bias.reshape(1, -1)
    idx2d = input_ids.astype(jnp.int32)
    tt3 = token_type_ids.reshape(b, s, 1)

    gathered = []
    base = 0
    for s_ch in _SC_SPLITS:
        g_h = _sc_gather_chunk(word_table, idx2d, base, b * s_ch, s, s_ch)
        gathered.append((base, s_ch, g_h.reshape(b, s_ch, _HIDDEN)))
        base += s_ch

    buf = None
    out_start = 0
    for size in _TC_SPLITS:
        sc_start, sc_size, g_h = next(
            (st, sz, g) for (st, sz, g) in gathered
            if st <= out_start and out_start + size <= st + sz
        )
        buf = _tc_add_ln_chunk(
            g_h, pos_table, tt3, type_table, ln_w, ln_b, buf,
            out_start - sc_start, out_start, size, b, s,
        )
        out_start += size
    return buf


# speedup vs baseline: 1.4002x; 1.0034x over previous
import functools

import jax
import jax.numpy as jnp
from jax import lax
from jax.experimental import pallas as pl
from jax.experimental.pallas import tpu as pltpu
from jax.experimental.pallas import tpu_sc as plsc

_HIDDEN = 768
_EPS = 1e-12
_NC = 2
_NS = 16
_NW = _NC * _NS
_CHUNK = 128
_SC_SPLITS = (1024, 1024)
_TC_SPLITS = (512, 512, 1024)


def _sc_gather_chunk(word_table, idx2d, chunk_base, n_rows, s, s_ch):
    per_w = n_rows // _NW
    n_sub = per_w // _CHUNK
    n_buf = min(2, n_sub)
    w_per_b = s_ch // per_w
    mesh = plsc.VectorSubcoreMesh(core_axis_name="c", subcore_axis_name="s")

    @functools.partial(
        pl.kernel,
        out_type=jax.ShapeDtypeStruct((n_rows, _HIDDEN), jnp.float32),
        mesh=mesh,
        scratch_types=(
            [pltpu.VMEM((per_w,), jnp.int32)]
            + [pltpu.VMEM((_CHUNK, _HIDDEN), jnp.float32)] * n_buf
            + [pltpu.SemaphoreType.DMA] * n_buf
        ),
    )
    def gather_kernel(table_hbm, idx_hbm, out_hbm, idx_v, *bufs_sems):
        bufs, sems = bufs_sems[:n_buf], bufs_sems[n_buf:]
        wid = lax.axis_index("s") * _NC + lax.axis_index("c")
        b = wid // w_per_b
        j = (wid % w_per_b) * per_w
        pltpu.sync_copy(idx_hbm.at[b, pl.ds(chunk_base + j, per_w)], idx_v)

        base = wid * per_w
        copies = [None] * n_sub
        copies[0] = pltpu.async_copy(
            table_hbm.at[idx_v.at[pl.ds(0, _CHUNK)]], bufs[0], sems[0]
        )
        for c in range(n_sub):
            if c + 1 < n_sub:
                copies[c + 1] = pltpu.async_copy(
                    table_hbm.at[idx_v.at[pl.ds((c + 1) * _CHUNK, _CHUNK)]],
                    bufs[(c + 1) % n_buf],
                    sems[(c + 1) % n_buf],
                )
            copies[c].wait()
            pltpu.sync_copy(bufs[c % n_buf], out_hbm.at[pl.ds(base + c * _CHUNK, _CHUNK)])

    return gather_kernel(word_table, idx2d)


def _ln_body(g_ref, p_ref, tt_ref, ty_ref, w_ref, b_ref, o_ref):
    x = g_ref[0] + p_ref[...]
    t0 = ty_ref[0:1, :]
    t1 = ty_ref[1:2, :]
    tt = tt_ref[0].astype(jnp.float32)
    x = x + t0 + tt * (t1 - t0)
    mean = jnp.mean(x, axis=1, keepdims=True)
    xc = x - mean
    var = jnp.mean(xc * xc, axis=1, keepdims=True)
    y = xc * lax.rsqrt(var + _EPS)
    o_ref[0] = y * w_ref[...] + b_ref[...]


def _ln_body_acc(g_ref, p_ref, tt_ref, ty_ref, w_ref, b_ref, _buf_ref, o_ref):
    _ln_body(g_ref, p_ref, tt_ref, ty_ref, w_ref, b_ref, o_ref)


def _tc_add_ln_chunk(g_h, pos_table, tt3, type_table, ln_w, ln_b, buf,
                     in_start, out_start, size, b, s):
    tblk = min(size, 1024)
    n_blk = size // tblk
    qin = in_start // tblk
    qout = out_start // tblk
    grid = (n_blk, b)
    in_specs = [
        pl.BlockSpec((1, tblk, _HIDDEN), lambda i, j, q=qin: (j, q + i, 0)),
        pl.BlockSpec((tblk, _HIDDEN), lambda i, j, q=qout: (q + i, 0)),
        pl.BlockSpec((1, tblk, 1), lambda i, j, q=qout: (j, q + i, 0)),
        pl.BlockSpec((2, _HIDDEN), lambda i, j: (0, 0)),
        pl.BlockSpec((1, _HIDDEN), lambda i, j: (0, 0)),
        pl.BlockSpec((1, _HIDDEN), lambda i, j: (0, 0)),
    ]
    args = [g_h, pos_table, tt3, type_table, ln_w, ln_b]
    body = _ln_body
    aliases = {}
    if buf is not None:
        in_specs.append(pl.BlockSpec(memory_space=pltpu.MemorySpace.HBM))
        args.append(buf)
        aliases = {6: 0}
        body = _ln_body_acc
    return pl.pallas_call(
        body,
        grid=grid,
        in_specs=in_specs,
        out_specs=pl.BlockSpec(
            (1, tblk, _HIDDEN), lambda i, j, q=qout: (j, q + i, 0)),
        out_shape=jax.ShapeDtypeStruct((b, s, _HIDDEN), jnp.float32),
        input_output_aliases=aliases,
    )(*args)


def kernel(input_ids, token_type_ids, word_table, pos_table, type_table, ln_weight, ln_bias):
    b, s = input_ids.shape
    ln_w = ln_weight.reshape(1, -1)
    ln_b = ln_bias.reshape(1, -1)
    idx2d = input_ids.astype(jnp.int32)
    tt3 = token_type_ids.reshape(b, s, 1)

    gathered = []
    base = 0
    for s_ch in _SC_SPLITS:
        g_h = _sc_gather_chunk(word_table, idx2d, base, b * s_ch, s, s_ch)
        gathered.append((base, s_ch, g_h.reshape(b, s_ch, _HIDDEN)))
        base += s_ch

    buf = None
    out_start = 0
    for size in _TC_SPLITS:
        sc_start, sc_size, g_h = next(
            (st, sz, g) for (st, sz, g) in gathered
            if st <= out_start and out_start + size <= st + sz
        )
        buf = _tc_add_ln_chunk(
            g_h, pos_table, tt3, type_table, ln_w, ln_b, buf,
            out_start - sc_start, out_start, size, b, s,
        )
        out_start += size
    return buf
